# Initial kernel scaffold; baseline (speedup 1.0000x reference)
import jax
import jax.numpy as jnp
from jax.experimental import pallas as pl


def kernel(x, edge_index, W1, b1, W2, b2, a):
    raise NotImplementedError("write your pallas kernel here")


# speedup vs baseline: 29.2399x; 29.2399x over previous
import functools

import jax
import jax.numpy as jnp
from jax import lax
from jax.experimental import pallas as pl
from jax.experimental.pallas import tpu as pltpu
from jax.experimental.pallas import tpu_sc as plsc

N_NODES = 10000
D = 128
N_EDGES = 320000

NC = 2
NS = 16
LANES = 16
NW = NC * NS
EPW = N_EDGES // NW

BM = 2048


def _tc_body(x_ref, w1_ref, b1_ref, w2_ref, b2_ref, a_ref, out_ref):
    xb = x_ref[...]
    h1 = lax.dot_general(xb, w1_ref[...], (((1,), (1,)), ((), ())),
                         preferred_element_type=jnp.float32)
    h1 = jnp.maximum(h1 + b1_ref[...], 0.0)
    h2 = lax.dot_general(h1, w2_ref[...], (((1,), (1,)), ((), ())),
                         preferred_element_type=jnp.float32)
    h2 = h2 + b2_ref[...]
    out_ref[...] = lax.dot_general(a_ref[...], h2, (((1,), (1,)), ((), ())),
                                   preferred_element_type=jnp.float32)


def _node_scores(x, W1, b1, W2, b2, a_resh):
    return pl.pallas_call(
        _tc_body,
        grid=(pl.cdiv(N_NODES, BM),),
        in_specs=[
            pl.BlockSpec((BM, D), lambda i: (i, 0)),
            pl.BlockSpec((D, D), lambda i: (0, 0)),
            pl.BlockSpec((1, D), lambda i: (0, 0)),
            pl.BlockSpec((D, D), lambda i: (0, 0)),
            pl.BlockSpec((1, D), lambda i: (0, 0)),
            pl.BlockSpec((2, D), lambda i: (0, 0)),
        ],
        out_specs=pl.BlockSpec((2, BM), lambda i: (0, i)),
        out_shape=jax.ShapeDtypeStruct((2, N_NODES), jnp.float32),
    )(x, W1, b1, W2, b2, a_resh)


_sc_mesh = plsc.VectorSubcoreMesh(core_axis_name="c", subcore_axis_name="s")


@functools.partial(
    pl.kernel,
    mesh=_sc_mesh,
    compiler_params=pltpu.CompilerParams(needs_layout_passes=False),
    out_type=jax.ShapeDtypeStruct((N_EDGES,), jnp.float32),
    scratch_types=[
        pltpu.VMEM((2 * N_NODES,), jnp.float32),
        pltpu.VMEM((EPW,), jnp.int32),
        pltpu.VMEM((EPW,), jnp.int32),
        pltpu.VMEM((EPW,), jnp.float32),
    ],
)
def _sc_edge(tab_hbm, src_hbm, dst_hbm, out_hbm, tab_v, src_v, dst_v, res_v):
    wid = lax.axis_index("s") * NC + lax.axis_index("c")
    base = wid * EPW
    pltpu.sync_copy(tab_hbm, tab_v)
    pltpu.sync_copy(src_hbm.at[pl.ds(base, EPW)], src_v)
    pltpu.sync_copy(dst_hbm.at[pl.ds(base, EPW)], dst_v)

    def body(i, carry):
        sl = pl.ds(i * LANES, LANES)
        isrc = src_v[sl]
        idst = dst_v[sl] + N_NODES
        v = plsc.load_gather(tab_v, [isrc]) + plsc.load_gather(tab_v, [idst])
        t = jnp.exp(-2.0 * jnp.abs(v))
        r = (1.0 - t) / (1.0 + t)
        res_v[sl] = jnp.where(v < 0.0, -r, r)
        return carry

    lax.fori_loop(0, EPW // LANES, body, 0)
    pltpu.sync_copy(res_v, out_hbm.at[pl.ds(base, EPW)])


def kernel(x, edge_index, W1, b1, W2, b2, a):
    s = _node_scores(x, W1, b1.reshape(1, D), W2, b2.reshape(1, D),
                     a.reshape(2, D))
    return _sc_edge(s.reshape(-1), edge_index[0], edge_index[1])


# speedup vs baseline: 39.2414x; 1.3420x over previous
import functools

import jax
import jax.numpy as jnp
from jax import lax
from jax.experimental import pallas as pl
from jax.experimental.pallas import tpu as pltpu
from jax.experimental.pallas import tpu_sc as plsc

N_NODES = 10000
D = 128
N_EDGES = 320000

NC = 2
NS = 16
LANES = 16
NW = NC * NS
EPW = N_EDGES // NW

BM = 2048


def _tc_body(x_ref, w1_ref, b1_ref, w2_ref, b2_ref, a_ref, out_ref):
    xb = x_ref[...]
    h1 = lax.dot_general(xb, w1_ref[...], (((1,), (1,)), ((), ())),
                         preferred_element_type=jnp.float32)
    h1 = jnp.maximum(h1 + b1_ref[...], 0.0)
    h2 = lax.dot_general(h1, w2_ref[...], (((1,), (1,)), ((), ())),
                         preferred_element_type=jnp.float32)
    h2 = h2 + b2_ref[...]
    out_ref[...] = lax.dot_general(a_ref[...], h2, (((1,), (1,)), ((), ())),
                                   preferred_element_type=jnp.float32)


def _node_scores(x, W1, b1, W2, b2, a_resh):
    return pl.pallas_call(
        _tc_body,
        grid=(pl.cdiv(N_NODES, BM),),
        in_specs=[
            pl.BlockSpec((BM, D), lambda i: (i, 0)),
            pl.BlockSpec((D, D), lambda i: (0, 0)),
            pl.BlockSpec((1, D), lambda i: (0, 0)),
            pl.BlockSpec((D, D), lambda i: (0, 0)),
            pl.BlockSpec((1, D), lambda i: (0, 0)),
            pl.BlockSpec((2, D), lambda i: (0, 0)),
        ],
        out_specs=pl.BlockSpec((2, BM), lambda i: (0, i)),
        out_shape=jax.ShapeDtypeStruct((2, N_NODES), jnp.float32),
    )(x, W1, b1, W2, b2, a_resh)


_sc_mesh = plsc.VectorSubcoreMesh(core_axis_name="c", subcore_axis_name="s")


@functools.partial(
    pl.kernel,
    mesh=_sc_mesh,
    compiler_params=pltpu.CompilerParams(needs_layout_passes=False),
    out_type=jax.ShapeDtypeStruct((N_EDGES,), jnp.float32),
    scratch_types=[
        pltpu.VMEM((N_NODES,), jnp.float32),
        pltpu.VMEM((N_NODES,), jnp.float32),
        pltpu.VMEM((EPW,), jnp.int32),
        pltpu.VMEM((EPW,), jnp.int32),
        pltpu.VMEM((EPW,), jnp.float32),
        pltpu.SemaphoreType.DMA,
    ],
)
def _sc_edge(tab_hbm, src_hbm, dst_hbm, out_hbm, s1_v, s2_v, src_v, dst_v,
             res_v, sem):
    wid = lax.axis_index("s") * NC + lax.axis_index("c")
    base = wid * EPW
    c1 = pltpu.async_copy(tab_hbm.at[pl.ds(0, N_NODES)], s1_v, sem)
    c2 = pltpu.async_copy(tab_hbm.at[pl.ds(N_NODES, N_NODES)], s2_v, sem)
    c3 = pltpu.async_copy(src_hbm.at[pl.ds(base, EPW)], src_v, sem)
    c4 = pltpu.async_copy(dst_hbm.at[pl.ds(base, EPW)], dst_v, sem)
    c1.wait()
    c2.wait()
    c3.wait()
    c4.wait()

    @plsc.parallel_loop(0, EPW, step=LANES, unroll=8)
    def body(i):
        sl = pl.ds(i, LANES)
        v = (plsc.load_gather(s1_v, [src_v[sl]])
             + plsc.load_gather(s2_v, [dst_v[sl]]))
        t = jnp.exp(-2.0 * jnp.abs(v))
        r = (1.0 - t) / (1.0 + t)
        res_v[sl] = jnp.where(v < 0.0, -r, r)

    pltpu.sync_copy(res_v, out_hbm.at[pl.ds(base, EPW)])


def kernel(x, edge_index, W1, b1, W2, b2, a):
    s = _node_scores(x, W1, b1.reshape(1, D), W2, b2.reshape(1, D),
                     a.reshape(2, D))
    return _sc_edge(s.reshape(-1), edge_index[0], edge_index[1])


# speedup vs baseline: 56.2885x; 1.4344x over previous
import functools

import jax
import jax.numpy as jnp
from jax import lax
from jax.experimental import pallas as pl
from jax.experimental.pallas import tpu as pltpu
from jax.experimental.pallas import tpu_sc as plsc

N_NODES = 10000
D = 128
N_EDGES = 320000

NC = 2
NS = 16
LANES = 16
NW = NC * NS
EPW = N_EDGES // NW

BM = 2048


def _tc_body(x_ref, w1_ref, b1_ref, w2_ref, b2_ref, a_ref, out_ref):
    xb = x_ref[...]
    h1 = lax.dot_general(xb, w1_ref[...], (((1,), (1,)), ((), ())),
                         preferred_element_type=jnp.float32)
    h1 = jnp.maximum(h1 + b1_ref[...], 0.0)
    h2 = lax.dot_general(h1, w2_ref[...], (((1,), (1,)), ((), ())),
                         preferred_element_type=jnp.float32)
    h2 = h2 + b2_ref[...]
    out_ref[...] = lax.dot_general(a_ref[...], h2, (((1,), (1,)), ((), ())),
                                   preferred_element_type=jnp.float32)


def _node_scores(x, W1, b1, W2, b2, a_resh):
    return pl.pallas_call(
        _tc_body,
        grid=(pl.cdiv(N_NODES, BM),),
        in_specs=[
            pl.BlockSpec((BM, D), lambda i: (i, 0)),
            pl.BlockSpec((D, D), lambda i: (0, 0)),
            pl.BlockSpec((1, D), lambda i: (0, 0)),
            pl.BlockSpec((D, D), lambda i: (0, 0)),
            pl.BlockSpec((1, D), lambda i: (0, 0)),
            pl.BlockSpec((2, D), lambda i: (0, 0)),
        ],
        out_specs=pl.BlockSpec((2, BM), lambda i: (0, i)),
        out_shape=jax.ShapeDtypeStruct((2, N_NODES), jnp.float32),
    )(x, W1, b1, W2, b2, a_resh)


_sc_mesh = plsc.VectorSubcoreMesh(core_axis_name="c", subcore_axis_name="s")


@functools.partial(
    pl.kernel,
    mesh=_sc_mesh,
    compiler_params=pltpu.CompilerParams(needs_layout_passes=False),
    out_type=jax.ShapeDtypeStruct((N_EDGES,), jnp.float32),
    scratch_types=[
        pltpu.VMEM((2, N_NODES), jnp.float32),
        pltpu.VMEM((2, EPW + 240), jnp.int32),
        pltpu.VMEM((EPW,), jnp.float32),
        pltpu.SemaphoreType.DMA,
    ],
)
def _sc_edge(tab_hbm, edge_hbm, out_hbm, tab_v, sd_v, res_v, sem):
    wid = lax.axis_index("s") * NC + lax.axis_index("c")
    base = wid * EPW
    padw = EPW + 240
    abase = jnp.minimum((base // 128) * 128, N_EDGES - padw)
    off = base - abase
    c1 = pltpu.async_copy(tab_hbm, tab_v, sem)
    c3 = pltpu.async_copy(edge_hbm.at[:, pl.ds(abase, padw)], sd_v, sem)
    c1.wait()
    c3.wait()

    zero16 = jnp.zeros((LANES,), jnp.int32)
    one16 = jnp.ones((LANES,), jnp.int32)

    @plsc.parallel_loop(0, EPW, step=LANES, unroll=8)
    def body(i):
        sl_in = pl.ds(off + i, LANES)
        v = (plsc.load_gather(tab_v, [zero16, sd_v[0, sl_in]])
             + plsc.load_gather(tab_v, [one16, sd_v[1, sl_in]]))
        t = jnp.exp(-2.0 * jnp.abs(v))
        r = (1.0 - t) / (1.0 + t)
        res_v[pl.ds(i, LANES)] = jnp.where(v < 0.0, -r, r)

    pltpu.sync_copy(res_v, out_hbm.at[pl.ds(base, EPW)])


def kernel(x, edge_index, W1, b1, W2, b2, a):
    s = _node_scores(x, W1, b1.reshape(1, D), W2, b2.reshape(1, D),
                     a.reshape(2, D))
    return _sc_edge(s, edge_index)
